# baseline (speedup 1.0000x reference)
import functools

import jax
import jax.numpy as jnp
from jax import lax
from jax.experimental import pallas as pl
from jax.experimental.pallas import tpu as pltpu
from jax.experimental.pallas import tpu_sc as plsc

_N = 32768
_B = 16
_DN = 256
_DM = 256
_NC = 2
_NS = 16
_NW = _NC * _NS
_ROWS = _N // _NW
_CH = 128
_NCHUNK = _ROWS // _CH
_G = _CH // 16


def _tree_sum(xs):
    while len(xs) > 1:
        xs = [a + b for a, b in zip(xs[::2], xs[1::2])]
    return xs[0]


def _sc_body(x_hbm, mr_hbm, wb_hbm, idx_hbm, out_hbm,
             xb0, xb1, w_v, mr_v, ms_v, idx_v, out_v, rot_v, pacc_v,
             sem0, sem1, sem_s):
    wid = lax.axis_index("s") * _NC + lax.axis_index("c")
    row0 = wid * _ROWS
    lanes = lax.iota(jnp.int32, 16)

    cp_w = pltpu.async_copy(wb_hbm, w_v, sem_s)
    cp_mr = pltpu.async_copy(mr_hbm, mr_v, sem_s)
    cp_idx = pltpu.async_copy(idx_hbm.at[pl.ds(row0, _ROWS)], idx_v, sem_s)
    bufs = [xb0, xb1]
    sems = [sem0, sem1]
    pltpu.async_copy(x_hbm.at[pl.ds(row0, _CH)], xb0, sem0)
    cp_w.wait()
    cp_mr.wait()
    cp_idx.wait()

    for j in range(16):
        rot_v[pl.ds(j * 16, 16)] = lanes * 16 + ((lanes + j) & 15)

    def diag_reduce(base):
        terms = []
        for j in range(16):
            rot = rot_v[pl.ds(j * 16, 16)]
            terms.append(plsc.load_gather(pacc_v, [base + rot]))
        return _tree_sum(terms)

    def row_partials(ref, r, wregs):
        prods = [ref[r, pl.ds(k * 16, 16)] * wregs[k] for k in range(16)]
        return _tree_sum(prods)

    w2 = [w_v[pl.ds(_DN + k * 16, 16)] for k in range(16)]
    for m in range(_B):
        pacc_v[pl.ds(m * 16, 16)] = row_partials(mr_v, m, w2)
    ms_v[...] = diag_reduce(0) + w_v[pl.ds(_DN + _DM, 16)]

    w1 = [w_v[pl.ds(k * 16, 16)] for k in range(16)]

    def do_chunk(ch, buf):
        def trbody(tr):
            for s in range(8):
                pacc_v[pl.ds(tr * 128 + s * 16, 16)] = row_partials(
                    buf, tr * 8 + s, w1)

        plsc.parallel_loop(0, _CH // 8)(trbody)

        def ebody(g):
            a = diag_reduce(g * 256)
            base = ch * _CH + g * 16
            seg = idx_v[pl.ds(base, 16)]
            logit = a + plsc.load_gather(ms_v, [seg])
            out_v[pl.ds(base, 16)] = 1.0 / (1.0 + jnp.exp(-logit))

        plsc.parallel_loop(0, _G)(ebody)

    def pair_body(p, _):
        for half in range(2):
            ch = p * 2 + half
            pltpu.make_async_copy(
                x_hbm.at[pl.ds(row0, _CH)], bufs[half],
                sems[half]).wait()
            do_chunk(ch, bufs[half])
            nxt = ch + 2

            @pl.when(nxt < _NCHUNK)
            def _():
                pltpu.async_copy(
                    x_hbm.at[pl.ds(row0 + nxt * _CH, _CH)],
                    bufs[half], sems[half])

        return 0

    pltpu.async_copy(x_hbm.at[pl.ds(row0 + _CH, _CH)], xb1, sem1)
    lax.fori_loop(0, _NCHUNK // 2, pair_body, 0)

    pltpu.sync_copy(out_v, out_hbm.at[pl.ds(row0, _ROWS)])


@jax.jit
def kernel(node_hiddens, next_motif_mreprs, W, b, batch_indices):
    wb = jnp.concatenate([W[:, 0], jnp.full((16,), b[0], jnp.float32)])
    mesh = plsc.VectorSubcoreMesh(core_axis_name="c", subcore_axis_name="s")
    run = pl.kernel(
        _sc_body,
        out_type=jax.ShapeDtypeStruct((_N,), jnp.float32),
        mesh=mesh,
        scratch_types=[
            pltpu.VMEM((_CH, _DN), jnp.float32),
            pltpu.VMEM((_CH, _DN), jnp.float32),
            pltpu.VMEM((_DN + _DM + 16,), jnp.float32),
            pltpu.VMEM((_B, _DM), jnp.float32),
            pltpu.VMEM((_B,), jnp.float32),
            pltpu.VMEM((_ROWS,), jnp.int32),
            pltpu.VMEM((_ROWS,), jnp.float32),
            pltpu.VMEM((16 * 16,), jnp.int32),
            pltpu.VMEM((_CH * 16,), jnp.float32),
            pltpu.SemaphoreType.DMA,
            pltpu.SemaphoreType.DMA,
            pltpu.SemaphoreType.DMA,
        ],
        compiler_params=pltpu.CompilerParams(
            use_tc_tiling_on_sc=True, needs_layout_passes=False,
            skip_device_barrier=True),
    )
    return run(node_hiddens, next_motif_mreprs, wb, batch_indices)


# speedup vs baseline: 1.0045x; 1.0045x over previous
import functools

import jax
import jax.numpy as jnp
from jax import lax
from jax.experimental import pallas as pl
from jax.experimental.pallas import tpu as pltpu
from jax.experimental.pallas import tpu_sc as plsc

_N = 32768
_B = 16
_DN = 256
_DM = 256
_NC = 2
_NS = 16
_NW = _NC * _NS
_ROWS = _N // _NW
_CH = 128
_NCHUNK = _ROWS // _CH
_G = _CH // 16


def _tree_sum(xs):
    while len(xs) > 1:
        xs = [a + b for a, b in zip(xs[::2], xs[1::2])]
    return xs[0]


def _sc_body(x_hbm, mr_hbm, wb_hbm, idx_hbm, out_hbm,
             xb0, xb1, w_v, mr_v, ms_v, idx_v, out_v, rot_v, pacc_v,
             sem0, sem1, sem_s):
    wid = lax.axis_index("s") * _NC + lax.axis_index("c")
    row0 = wid * _ROWS
    lanes = lax.iota(jnp.int32, 16)

    cp_w = pltpu.async_copy(wb_hbm, w_v, sem_s)
    cp_mr = pltpu.async_copy(mr_hbm, mr_v, sem_s)
    cp_idx = pltpu.async_copy(idx_hbm.at[pl.ds(row0, _ROWS)], idx_v, sem_s)
    bufs = [xb0, xb1]
    sems = [sem0, sem1]
    pltpu.async_copy(x_hbm.at[pl.ds(row0, _CH)], xb0, sem0)
    cp_w.wait()
    cp_mr.wait()
    cp_idx.wait()

    for j in range(16):
        rot_v[pl.ds(j * 16, 16)] = lanes * 16 + ((lanes + j) & 15)

    def diag_reduce(base):
        terms = []
        for j in range(16):
            rot = rot_v[pl.ds(j * 16, 16)]
            terms.append(plsc.load_gather(pacc_v, [base + rot]))
        return _tree_sum(terms)

    def row_partials(ref, r, wregs):
        prods = [ref[r, pl.ds(k * 16, 16)] * wregs[k] for k in range(16)]
        return _tree_sum(prods)

    w2 = [w_v[pl.ds(_DN + k * 16, 16)] for k in range(16)]
    for m in range(_B):
        pacc_v[pl.ds(m * 16, 16)] = row_partials(mr_v, m, w2)
    ms_v[...] = diag_reduce(0) + w_v[pl.ds(_DN + _DM, 16)]

    w1 = [w_v[pl.ds(k * 16, 16)] for k in range(16)]

    def do_chunk(ch, buf):
        def trbody(tr):
            for s in range(8):
                pacc_v[pl.ds(tr * 128 + s * 16, 16)] = row_partials(
                    buf, tr * 8 + s, w1)

        plsc.parallel_loop(0, _CH // 8, unroll=2)(trbody)

        def ebody(g):
            a = diag_reduce(g * 256)
            base = ch * _CH + g * 16
            seg = idx_v[pl.ds(base, 16)]
            logit = a + plsc.load_gather(ms_v, [seg])
            out_v[pl.ds(base, 16)] = 1.0 / (1.0 + jnp.exp(-logit))

        plsc.parallel_loop(0, _G)(ebody)

    def pair_body(p, _):
        for half in range(2):
            ch = p * 2 + half
            pltpu.make_async_copy(
                x_hbm.at[pl.ds(row0, _CH)], bufs[half],
                sems[half]).wait()
            do_chunk(ch, bufs[half])
            nxt = ch + 2

            @pl.when(nxt < _NCHUNK)
            def _():
                pltpu.async_copy(
                    x_hbm.at[pl.ds(row0 + nxt * _CH, _CH)],
                    bufs[half], sems[half])

        return 0

    pltpu.async_copy(x_hbm.at[pl.ds(row0 + _CH, _CH)], xb1, sem1)
    lax.fori_loop(0, _NCHUNK // 2, pair_body, 0)

    pltpu.sync_copy(out_v, out_hbm.at[pl.ds(row0, _ROWS)])


@jax.jit
def kernel(node_hiddens, next_motif_mreprs, W, b, batch_indices):
    wb = jnp.concatenate([W[:, 0], jnp.full((16,), b[0], jnp.float32)])
    mesh = plsc.VectorSubcoreMesh(core_axis_name="c", subcore_axis_name="s")
    run = pl.kernel(
        _sc_body,
        out_type=jax.ShapeDtypeStruct((_N,), jnp.float32),
        mesh=mesh,
        scratch_types=[
            pltpu.VMEM((_CH, _DN), jnp.float32),
            pltpu.VMEM((_CH, _DN), jnp.float32),
            pltpu.VMEM((_DN + _DM + 16,), jnp.float32),
            pltpu.VMEM((_B, _DM), jnp.float32),
            pltpu.VMEM((_B,), jnp.float32),
            pltpu.VMEM((_ROWS,), jnp.int32),
            pltpu.VMEM((_ROWS,), jnp.float32),
            pltpu.VMEM((16 * 16,), jnp.int32),
            pltpu.VMEM((_CH * 16,), jnp.float32),
            pltpu.SemaphoreType.DMA,
            pltpu.SemaphoreType.DMA,
            pltpu.SemaphoreType.DMA,
        ],
        compiler_params=pltpu.CompilerParams(
            use_tc_tiling_on_sc=True, needs_layout_passes=False),
    )
    return run(node_hiddens, next_motif_mreprs, wb, batch_indices)
